# baseline (speedup 1.0000x reference)
import jax
import jax.numpy as jnp
from jax.experimental import pallas as pl

_STEPS_PAD = 1024
_BB = 32


def _scale_kernel(ts_ref, a_ref, s_ref, x_ref, n_ref, o_ref):
    t = ts_ref[0, 0, :]
    iota = jax.lax.broadcasted_iota(jnp.int32, (_BB, _STEPS_PAD), 1)
    onehot = iota == t[:, None]
    ca = jnp.sum(jnp.where(onehot, a_ref[0, :][None, :], 0.0), axis=1)
    cs = jnp.sum(jnp.where(onehot, s_ref[0, :][None, :], 0.0), axis=1)
    o_ref[...] = (ca[:, None, None, None] * x_ref[...]
                  + cs[:, None, None, None] * n_ref[...])


def kernel(x0, noise, timesteps, sqrt_alphas_cumprod, sqrt_one_minus_alphas_cumprod):
    B, C, H, W = x0.shape
    nb = B // _BB
    ts3 = timesteps.reshape(nb, 1, _BB)
    steps = sqrt_alphas_cumprod.shape[0]
    a_p = jnp.zeros((1, _STEPS_PAD), x0.dtype).at[0, :steps].set(sqrt_alphas_cumprod)
    s_p = jnp.zeros((1, _STEPS_PAD), x0.dtype).at[0, :steps].set(
        sqrt_one_minus_alphas_cumprod)

    out = pl.pallas_call(
        _scale_kernel,
        grid=(nb,),
        in_specs=[
            pl.BlockSpec((1, 1, _BB), lambda i: (i, 0, 0)),
            pl.BlockSpec((1, _STEPS_PAD), lambda i: (0, 0)),
            pl.BlockSpec((1, _STEPS_PAD), lambda i: (0, 0)),
            pl.BlockSpec((_BB, C, H, W), lambda i: (i, 0, 0, 0)),
            pl.BlockSpec((_BB, C, H, W), lambda i: (i, 0, 0, 0)),
        ],
        out_specs=pl.BlockSpec((_BB, C, H, W), lambda i: (i, 0, 0, 0)),
        out_shape=jax.ShapeDtypeStruct((B, C, H, W), x0.dtype),
    )(ts3, a_p, s_p, x0, noise)
    return out


# speedup vs baseline: 7.0927x; 7.0927x over previous
import jax
import jax.numpy as jnp
from jax.experimental import pallas as pl
from jax.experimental.pallas import tpu as pltpu

_STEPS_PAD = 1024
_FB = 256
_KC = 256


def _scale_kernel(ts_ref, tab_ref, x_ref, n_ref, o_ref, coef_ref):
    @pl.when(pl.program_id(0) == 0)
    def _():
        t = ts_ref[...]
        acc = jnp.zeros((2, t.shape[1]), jnp.float32)
        for kc in range(_STEPS_PAD // _KC):
            rows = jax.lax.broadcasted_iota(
                jnp.int32, (_KC, t.shape[1]), 0) + kc * _KC
            onehot = (rows == t).astype(jnp.float32)
            acc = acc + jax.lax.dot_general(
                tab_ref[:, kc * _KC:(kc + 1) * _KC], onehot,
                (((1,), (0,)), ((), ())),
                preferred_element_type=jnp.float32)
        coef_ref[...] = acc

    ca = coef_ref[0:1, :]
    cs = coef_ref[1:2, :]
    o_ref[...] = ca * x_ref[...] + cs * n_ref[...]


def kernel(x0, noise, timesteps, sqrt_alphas_cumprod, sqrt_one_minus_alphas_cumprod):
    B, C, H, W = x0.shape
    F = C * H * W
    x = x0.transpose(1, 2, 3, 0).reshape(F, B)
    n = noise.transpose(1, 2, 3, 0).reshape(F, B)
    ts2 = timesteps.reshape(1, B)
    steps = sqrt_alphas_cumprod.shape[0]
    tab = jnp.zeros((2, _STEPS_PAD), jnp.float32)
    tab = tab.at[0, :steps].set(sqrt_alphas_cumprod)
    tab = tab.at[1, :steps].set(sqrt_one_minus_alphas_cumprod)

    out = pl.pallas_call(
        _scale_kernel,
        grid=(F // _FB,),
        in_specs=[
            pl.BlockSpec((1, B), lambda i: (0, 0)),
            pl.BlockSpec((2, _STEPS_PAD), lambda i: (0, 0)),
            pl.BlockSpec((_FB, B), lambda i: (i, 0)),
            pl.BlockSpec((_FB, B), lambda i: (i, 0)),
        ],
        out_specs=pl.BlockSpec((_FB, B), lambda i: (i, 0)),
        out_shape=jax.ShapeDtypeStruct((F, B), x0.dtype),
        scratch_shapes=[pltpu.VMEM((2, B), jnp.float32)],
    )(ts2, tab, x, n)
    return out.reshape(C, H, W, B).transpose(3, 0, 1, 2)
